# Initial kernel scaffold; baseline (speedup 1.0000x reference)
import jax
import jax.numpy as jnp
from jax.experimental import pallas as pl


def kernel(x, W1a, b1a, W1b, b1b, W2a, b2a, W2b, b2b, W3a, b3a, W3b, b3b, P1, pb1, P2, pb2, M1, mb1, M2, mb2, M3, mb3, batch):
    raise NotImplementedError("write your pallas kernel here")


# speedup vs baseline: 9.3149x; 9.3149x over previous
import functools

import jax
import jax.numpy as jnp
from jax import lax
from jax.experimental import pallas as pl
from jax.experimental.pallas import tpu as pltpu
from jax.experimental.pallas import tpu_sc as plsc

K = 16
NEG_INF = float("-inf")
IDX_SENTINEL = 2**30


def _knn_body(lo_ref, hi_ref, hrow_ref, hcol_ref, brow_ref, bcol_ref,
              wdiff_ref, wbot_ref, ba_ref,
              idx_ref, a_ref, g_ref, bestv, besti, *, R, CB, SLOTS):
    r = pl.program_id(0)
    hr = hrow_ref[...]
    br = brow_ref[...]

    bestv[...] = jnp.full((R, SLOTS), NEG_INF, jnp.float32)
    besti[...] = jnp.full((R, SLOTS), IDX_SENTINEL, jnp.int32)

    lo = lo_ref[r]
    hi = hi_ref[r]
    c0 = lo // CB
    c1 = (hi + (CB - 1)) // CB

    def col_step(c, carry):
        off = c * CB
        hc = hcol_ref[pl.ds(off, CB), :]
        bc = bcol_ref[:, pl.ds(off, CB)]
        dot = lax.dot_general(hr, hc, (((1,), (1,)), ((), ())),
                              preferred_element_type=jnp.float32)
        sqc = jnp.sum(hc * hc, axis=1)
        score = dot - 0.5 * sqc[None, :]
        valid = br == bc
        score = jnp.where(valid, score, NEG_INF)
        colidx = off + lax.broadcasted_iota(jnp.int32, (R, CB), 1)

        cv = jnp.concatenate([bestv[...], score], axis=1)
        ci = jnp.concatenate([besti[...], colidx], axis=1)
        for t in range(K):
            m = jnp.max(cv, axis=1, keepdims=True)
            ism = cv == m
            wi = jnp.min(jnp.where(ism, ci, IDX_SENTINEL), axis=1,
                         keepdims=True)
            hit = ism & (ci == wi)
            cv = jnp.where(hit, NEG_INF, cv)
            bestv[:, t:t + 1] = m
            besti[:, t:t + 1] = wi
        return carry

    lax.fori_loop(c0, c1, col_step, 0)
    idx_ref[...] = jnp.clip(besti[:, :K], 0, hcol_ref.shape[0] - 1)

    a_ref[...] = lax.dot_general(hr, wdiff_ref[...], (((1,), (0,)), ((), ())),
                                 preferred_element_type=jnp.float32) + ba_ref[...]
    g_ref[...] = lax.dot_general(hr, wbot_ref[...], (((1,), (0,)), ((), ())),
                                 preferred_element_type=jnp.float32)


def _knn_and_terms(h, brow, bcol, lo, hi, wdiff, wbot, ba, *, R=256, CB=256):
    N, D = h.shape
    Hh = wdiff.shape[1]
    nrb = N // R
    SLOTS = 128
    body = functools.partial(_knn_body, R=R, CB=CB, SLOTS=SLOTS)
    return pl.pallas_call(
        body,
        grid=(nrb,),
        in_specs=[
            pl.BlockSpec(memory_space=pltpu.SMEM),
            pl.BlockSpec(memory_space=pltpu.SMEM),
            pl.BlockSpec((R, D), lambda i: (i, 0)),
            pl.BlockSpec((N, D), lambda i: (0, 0)),
            pl.BlockSpec((R, 1), lambda i: (i, 0)),
            pl.BlockSpec((1, N), lambda i: (0, 0)),
            pl.BlockSpec((D, Hh), lambda i: (0, 0)),
            pl.BlockSpec((D, Hh), lambda i: (0, 0)),
            pl.BlockSpec((1, Hh), lambda i: (0, 0)),
        ],
        out_specs=[
            pl.BlockSpec((R, K), lambda i: (i, 0)),
            pl.BlockSpec((R, Hh), lambda i: (i, 0)),
            pl.BlockSpec((R, Hh), lambda i: (i, 0)),
        ],
        out_shape=[
            jax.ShapeDtypeStruct((N, K), jnp.int32),
            jax.ShapeDtypeStruct((N, Hh), jnp.float32),
            jax.ShapeDtypeStruct((N, Hh), jnp.float32),
        ],
        scratch_shapes=[
            pltpu.VMEM((R, SLOTS), jnp.float32),
            pltpu.VMEM((R, SLOTS), jnp.int32),
        ],
    )(lo, hi, h, h, brow, bcol, wdiff, wbot, ba)


def _sc_gather(table, idx2d):
    V, Hh = table.shape
    CH = idx2d.shape[1]
    E = idx2d.shape[0] * CH
    info = plsc.get_sparse_core_info()
    NW = info.num_cores * info.num_subcores
    per_w = E // NW
    nch = per_w // CH
    rows_per_w = per_w // CH

    mesh = plsc.VectorSubcoreMesh(core_axis_name="c", subcore_axis_name="s")

    @functools.partial(
        pl.kernel, mesh=mesh,
        out_type=jax.ShapeDtypeStruct((E, Hh), jnp.float32),
        scratch_types=[
            pltpu.VMEM((nch, CH), jnp.int32),
            pltpu.VMEM((CH, Hh), jnp.float32),
            pltpu.VMEM((CH, Hh), jnp.float32),
            pltpu.SemaphoreType.DMA,
            pltpu.SemaphoreType.DMA,
        ],
    )
    def gather_k(table_hbm, idx_hbm, out_hbm, idx_v, buf0, buf1, sem0, sem1):
        wid = lax.axis_index("s") * info.num_cores + lax.axis_index("c")
        base = wid * per_w
        pltpu.sync_copy(idx_hbm.at[pl.ds(wid * nch, nch)], idx_v)

        def start(j, buf, sem):
            return pltpu.async_copy(table_hbm.at[idx_v.at[j]], buf, sem)

        def wait(j, buf, sem):
            pltpu.make_async_copy(table_hbm.at[idx_v.at[j]], buf, sem).wait()

        start(0, buf0, sem0)

        def pair(p, carry):
            j0 = 2 * p
            j1 = j0 + 1
            start(j1, buf1, sem1)
            wait(j0, buf0, sem0)
            pltpu.sync_copy(buf0, out_hbm.at[pl.ds(base + j0 * CH, CH)])

            @pl.when(j1 + 1 < nch)
            def _():
                start(j1 + 1, buf0, sem0)

            wait(j1, buf1, sem1)
            pltpu.sync_copy(buf1, out_hbm.at[pl.ds(base + j1 * CH, CH)])
            return carry

        lax.fori_loop(0, nch // 2, pair, 0)

    return gather_k(table, idx2d)


def _conv_body(a_ref, xg_ref, wb_ref, bb_ref, out_ref, *, R, Hh):
    a = a_ref[...]
    wb = wb_ref[...]
    acc = jnp.full((R, Hh), NEG_INF, jnp.float32)
    for k in range(K):
        e = jnp.maximum(a + xg_ref[k], 0.0)
        mm = lax.dot_general(e, wb, (((1,), (0,)), ((), ())),
                             preferred_element_type=jnp.float32)
        acc = jnp.maximum(acc, mm)
    out_ref[...] = acc + bb_ref[...]


def _edge_conv(a, xg, wb, bb, *, R=256):
    N, Hh = a.shape
    nrb = N // R
    body = functools.partial(_conv_body, R=R, Hh=Hh)
    return pl.pallas_call(
        body,
        grid=(nrb,),
        in_specs=[
            pl.BlockSpec((R, Hh), lambda i: (i, 0)),
            pl.BlockSpec((K, R, Hh), lambda i: (0, i, 0)),
            pl.BlockSpec((Hh, Hh), lambda i: (0, 0)),
            pl.BlockSpec((1, Hh), lambda i: (0, 0)),
        ],
        out_specs=pl.BlockSpec((R, Hh), lambda i: (i, 0)),
        out_shape=jax.ShapeDtypeStruct((N, Hh), jnp.float32),
    )(a, xg, wb, bb)


def _elu(x):
    return jnp.where(x > 0, x, jnp.exp(x) - 1.0)


def _head1_body(h1_ref, h2_ref, h3_ref, brow_ref, p1a_ref, p1b_ref, p1c_ref,
                pb1_ref, p2_ref, pb2_ref, out_ref, *, R, Hh, NB):
    i = pl.program_id(0)

    @pl.when(i == 0)
    def _():
        out_ref[...] = jnp.full(out_ref.shape, NEG_INF, jnp.float32)

    z = (lax.dot_general(h1_ref[...], p1a_ref[...], (((1,), (0,)), ((), ())),
                         preferred_element_type=jnp.float32)
         + lax.dot_general(h2_ref[...], p1b_ref[...], (((1,), (0,)), ((), ())),
                           preferred_element_type=jnp.float32)
         + lax.dot_general(h3_ref[...], p1c_ref[...], (((1,), (0,)), ((), ())),
                           preferred_element_type=jnp.float32)
         + pb1_ref[...])
    z = _elu(z)
    z = _elu(lax.dot_general(z, p2_ref[...], (((1,), (0,)), ((), ())),
                             preferred_element_type=jnp.float32) + pb2_ref[...])
    b = brow_ref[...]
    for s in range(NB):
        m = b == float(s)
        v = jnp.where(m, z, NEG_INF)
        red = jnp.max(v, axis=0, keepdims=True)
        out_ref[s:s + 1, :] = jnp.maximum(out_ref[s:s + 1, :], red)


def _head1(h1, h2, h3, brow, p1a, p1b, p1c, pb1, p2, pb2, nb, *, R=512):
    N, Hh = h1.shape
    nrb = N // R
    body = functools.partial(_head1_body, R=R, Hh=Hh, NB=nb)
    return pl.pallas_call(
        body,
        grid=(nrb,),
        in_specs=[
            pl.BlockSpec((R, Hh), lambda i: (i, 0)),
            pl.BlockSpec((R, Hh), lambda i: (i, 0)),
            pl.BlockSpec((R, Hh), lambda i: (i, 0)),
            pl.BlockSpec((R, 1), lambda i: (i, 0)),
            pl.BlockSpec((Hh, Hh), lambda i: (0, 0)),
            pl.BlockSpec((Hh, Hh), lambda i: (0, 0)),
            pl.BlockSpec((Hh, Hh), lambda i: (0, 0)),
            pl.BlockSpec((1, Hh), lambda i: (0, 0)),
            pl.BlockSpec((Hh, Hh), lambda i: (0, 0)),
            pl.BlockSpec((1, Hh), lambda i: (0, 0)),
        ],
        out_specs=pl.BlockSpec((nb, Hh), lambda i: (0, 0)),
        out_shape=jax.ShapeDtypeStruct((nb, Hh), jnp.float32),
    )(h1, h2, h3, brow, p1a, p1b, p1c, pb1, p2, pb2)


def _head2_body(g_ref, m1_ref, mb1_ref, m2_ref, mb2_ref, m3_ref, mb3_ref,
                out_ref):
    g = _elu(lax.dot_general(g_ref[...], m1_ref[...], (((1,), (0,)), ((), ())),
                             preferred_element_type=jnp.float32) + mb1_ref[...])
    g = _elu(lax.dot_general(g, m2_ref[...], (((1,), (0,)), ((), ())),
                             preferred_element_type=jnp.float32) + mb2_ref[...])
    o = lax.dot_general(g, m3_ref[...], (((1,), (0,)), ((), ())),
                        preferred_element_type=jnp.float32) + mb3_ref[...]
    m = jnp.max(o, axis=1, keepdims=True)
    lse = jnp.log(jnp.sum(jnp.exp(o - m), axis=1, keepdims=True)) + m
    out_ref[...] = o - lse


def _head2(g, m1, mb1, m2, mb2, m3, mb3):
    nb, Hh = g.shape
    C = m3.shape[1]
    return pl.pallas_call(
        _head2_body,
        out_shape=jax.ShapeDtypeStruct((nb, C), jnp.float32),
    )(g, m1, mb1, m2, mb2, m3, mb3)


def kernel(x, W1a, b1a, W1b, b1b, W2a, b2a, W2b, b2b, W3a, b3a, W3b, b3b,
           P1, pb1, P2, pb2, M1, mb1, M2, mb2, M3, mb3, batch):
    N, D_in = x.shape
    Hh = W1b.shape[0]
    NB = 8
    R = 256

    bi = batch.astype(jnp.int32)
    seg_lo = jnp.searchsorted(bi, jnp.arange(NB, dtype=jnp.int32), side="left")
    seg_hi = jnp.searchsorted(bi, jnp.arange(NB, dtype=jnp.int32), side="right")
    lo = seg_lo[bi[::R]].astype(jnp.int32)
    hi = seg_hi[bi[R - 1::R]].astype(jnp.int32)

    bf = batch.astype(jnp.float32)
    brow = bf.reshape(N, 1)
    bcol = bf.reshape(1, N)

    Dp = 8
    xp = jnp.concatenate(
        [x, jnp.zeros((N, Dp - D_in), jnp.float32)], axis=1)

    def split_wa(Wa, d, dp):
        top, bot = Wa[:d], Wa[d:]
        diff = top - bot
        if dp > d:
            pad = jnp.zeros((dp - d, Wa.shape[1]), jnp.float32)
            diff = jnp.concatenate([diff, pad], axis=0)
            bot = jnp.concatenate([bot, pad], axis=0)
        return diff, bot

    h = xp
    d_cur, dp_cur = D_in, Dp
    hs = []
    for (Wa, ba, Wb, bb) in ((W1a, b1a, W1b, b1b),
                             (W2a, b2a, W2b, b2b),
                             (W3a, b3a, W3b, b3b)):
        wdiff, wbot = split_wa(Wa, d_cur, dp_cur)
        idx, a_t, g_t = _knn_and_terms(
            h, brow, bcol, lo, hi, wdiff, wbot, ba.reshape(1, Hh), R=R)
        idx_flat = idx.T.reshape(-1)
        xg = _sc_gather(g_t, idx_flat.reshape(-1, 128))
        xg = xg.reshape(K, N, Hh)
        h = _edge_conv(a_t, xg, Wb, bb.reshape(1, Hh), R=R)
        hs.append(h)
        d_cur = dp_cur = Hh

    p1a, p1b, p1c = P1[:Hh], P1[Hh:2 * Hh], P1[2 * Hh:]
    g = _head1(hs[0], hs[1], hs[2], brow, p1a, p1b, p1c,
               pb1.reshape(1, Hh), P2, pb2.reshape(1, Hh), NB)
    return _head2(g, M1, mb1.reshape(1, Hh), M2, mb2.reshape(1, Hh),
                  M3, mb3.reshape(1, M3.shape[1]))


# speedup vs baseline: 19.4974x; 2.0931x over previous
import functools

import jax
import jax.numpy as jnp
from jax import lax
from jax.experimental import pallas as pl
from jax.experimental.pallas import tpu as pltpu
from jax.experimental.pallas import tpu_sc as plsc

K = 16
NEG_INF = float("-inf")
IDX_SENTINEL = 2**30


def _knn_body(lo_ref, hi_ref, hrow_ref, hcol_ref, brow_ref, bcol_ref,
              wdiff_ref, wbot_ref, ba_ref,
              idx_ref, a_ref, g_ref, bestv, besti, *, R, CB):
    r = pl.program_id(0)
    hr = hrow_ref[...]
    brt = bcol_ref[:, pl.ds(r * R, R)]

    bestv[...] = jnp.full((K, R), NEG_INF, jnp.float32)
    besti[...] = jnp.full((K, R), IDX_SENTINEL, jnp.int32)

    lo = lo_ref[r]
    hi = hi_ref[r]
    c0 = lo // CB
    c1 = (hi + (CB - 1)) // CB

    def col_step(c, carry):
        off = c * CB
        hc = hcol_ref[pl.ds(off, CB), :]
        bc = brow_ref[pl.ds(off, CB), :]
        dot = lax.dot_general(hc, hr, (((1,), (1,)), ((), ())),
                              preferred_element_type=jnp.float32)
        sqc = jnp.sum(hc * hc, axis=1, keepdims=True)
        score = dot - 0.5 * sqc
        valid = bc == brt
        score = jnp.where(valid, score, NEG_INF)
        colidx = off + lax.broadcasted_iota(jnp.int32, (CB, R), 0)

        cv = jnp.concatenate([bestv[...], score], axis=0)
        ci = jnp.concatenate([besti[...], colidx], axis=0)
        for t in range(K):
            m = jnp.max(cv, axis=0, keepdims=True)
            ism = cv == m
            wi = jnp.min(jnp.where(ism, ci, IDX_SENTINEL), axis=0,
                         keepdims=True)
            hit = ism & (ci == wi)
            cv = jnp.where(hit, NEG_INF, cv)
            bestv[t:t + 1, :] = m
            besti[t:t + 1, :] = wi
        return carry

    lax.fori_loop(c0, c1, col_step, 0)
    idx_ref[...] = jnp.clip(besti[...], 0, hcol_ref.shape[0] - 1)

    a_ref[...] = lax.dot_general(hr, wdiff_ref[...], (((1,), (0,)), ((), ())),
                                 preferred_element_type=jnp.float32) + ba_ref[...]
    g_ref[...] = lax.dot_general(hr, wbot_ref[...], (((1,), (0,)), ((), ())),
                                 preferred_element_type=jnp.float32)


def _knn_and_terms(h, brow, bcol, lo, hi, wdiff, wbot, ba, *, R=256, CB=256):
    N, D = h.shape
    Hh = wdiff.shape[1]
    nrb = N // R
    body = functools.partial(_knn_body, R=R, CB=CB)
    return pl.pallas_call(
        body,
        grid=(nrb,),
        in_specs=[
            pl.BlockSpec(memory_space=pltpu.SMEM),
            pl.BlockSpec(memory_space=pltpu.SMEM),
            pl.BlockSpec((R, D), lambda i: (i, 0)),
            pl.BlockSpec((N, D), lambda i: (0, 0)),
            pl.BlockSpec((N, 1), lambda i: (0, 0)),
            pl.BlockSpec((1, N), lambda i: (0, 0)),
            pl.BlockSpec((D, Hh), lambda i: (0, 0)),
            pl.BlockSpec((D, Hh), lambda i: (0, 0)),
            pl.BlockSpec((1, Hh), lambda i: (0, 0)),
        ],
        out_specs=[
            pl.BlockSpec((K, R), lambda i: (0, i)),
            pl.BlockSpec((R, Hh), lambda i: (i, 0)),
            pl.BlockSpec((R, Hh), lambda i: (i, 0)),
        ],
        out_shape=[
            jax.ShapeDtypeStruct((K, N), jnp.int32),
            jax.ShapeDtypeStruct((N, Hh), jnp.float32),
            jax.ShapeDtypeStruct((N, Hh), jnp.float32),
        ],
        scratch_shapes=[
            pltpu.VMEM((K, R), jnp.float32),
            pltpu.VMEM((K, R), jnp.int32),
        ],
    )(lo, hi, h, h, brow, bcol, wdiff, wbot, ba)


def _sc_gather(table, idx2d):
    V, Hh = table.shape
    CH = idx2d.shape[1]
    E = idx2d.shape[0] * CH
    info = plsc.get_sparse_core_info()
    NW = info.num_cores * info.num_subcores
    per_w = E // NW
    nch = per_w // CH
    rows_per_w = per_w // CH

    mesh = plsc.VectorSubcoreMesh(core_axis_name="c", subcore_axis_name="s")

    @functools.partial(
        pl.kernel, mesh=mesh,
        out_type=jax.ShapeDtypeStruct((E, Hh), jnp.float32),
        scratch_types=[
            pltpu.VMEM((nch, CH), jnp.int32),
            pltpu.VMEM((CH, Hh), jnp.float32),
            pltpu.VMEM((CH, Hh), jnp.float32),
            pltpu.SemaphoreType.DMA,
            pltpu.SemaphoreType.DMA,
        ],
    )
    def gather_k(table_hbm, idx_hbm, out_hbm, idx_v, buf0, buf1, sem0, sem1):
        wid = lax.axis_index("s") * info.num_cores + lax.axis_index("c")
        base = wid * per_w
        pltpu.sync_copy(idx_hbm.at[pl.ds(wid * nch, nch)], idx_v)

        def start(j, buf, sem):
            return pltpu.async_copy(table_hbm.at[idx_v.at[j]], buf, sem)

        def wait(j, buf, sem):
            pltpu.make_async_copy(table_hbm.at[idx_v.at[j]], buf, sem).wait()

        start(0, buf0, sem0)

        def pair(p, carry):
            j0 = 2 * p
            j1 = j0 + 1
            start(j1, buf1, sem1)
            wait(j0, buf0, sem0)
            pltpu.sync_copy(buf0, out_hbm.at[pl.ds(base + j0 * CH, CH)])

            @pl.when(j1 + 1 < nch)
            def _():
                start(j1 + 1, buf0, sem0)

            wait(j1, buf1, sem1)
            pltpu.sync_copy(buf1, out_hbm.at[pl.ds(base + j1 * CH, CH)])
            return carry

        lax.fori_loop(0, nch // 2, pair, 0)

    return gather_k(table, idx2d)


def _conv_body(a_ref, xg_ref, wb_ref, bb_ref, out_ref, *, R, Hh):
    a = a_ref[...]
    wb = wb_ref[...]
    acc = jnp.full((R, Hh), NEG_INF, jnp.float32)
    for k in range(K):
        e = jnp.maximum(a + xg_ref[k], 0.0)
        mm = lax.dot_general(e, wb, (((1,), (0,)), ((), ())),
                             preferred_element_type=jnp.float32)
        acc = jnp.maximum(acc, mm)
    out_ref[...] = acc + bb_ref[...]


def _edge_conv(a, xg, wb, bb, *, R=256):
    N, Hh = a.shape
    nrb = N // R
    body = functools.partial(_conv_body, R=R, Hh=Hh)
    return pl.pallas_call(
        body,
        grid=(nrb,),
        in_specs=[
            pl.BlockSpec((R, Hh), lambda i: (i, 0)),
            pl.BlockSpec((K, R, Hh), lambda i: (0, i, 0)),
            pl.BlockSpec((Hh, Hh), lambda i: (0, 0)),
            pl.BlockSpec((1, Hh), lambda i: (0, 0)),
        ],
        out_specs=pl.BlockSpec((R, Hh), lambda i: (i, 0)),
        out_shape=jax.ShapeDtypeStruct((N, Hh), jnp.float32),
    )(a, xg, wb, bb)


def _elu(x):
    return jnp.where(x > 0, x, jnp.exp(x) - 1.0)


def _head1_body(h1_ref, h2_ref, h3_ref, brow_ref, p1a_ref, p1b_ref, p1c_ref,
                pb1_ref, p2_ref, pb2_ref, out_ref, *, R, Hh, NB):
    i = pl.program_id(0)

    @pl.when(i == 0)
    def _():
        out_ref[...] = jnp.full(out_ref.shape, NEG_INF, jnp.float32)

    z = (lax.dot_general(h1_ref[...], p1a_ref[...], (((1,), (0,)), ((), ())),
                         preferred_element_type=jnp.float32)
         + lax.dot_general(h2_ref[...], p1b_ref[...], (((1,), (0,)), ((), ())),
                           preferred_element_type=jnp.float32)
         + lax.dot_general(h3_ref[...], p1c_ref[...], (((1,), (0,)), ((), ())),
                           preferred_element_type=jnp.float32)
         + pb1_ref[...])
    z = _elu(z)
    z = _elu(lax.dot_general(z, p2_ref[...], (((1,), (0,)), ((), ())),
                             preferred_element_type=jnp.float32) + pb2_ref[...])
    b = brow_ref[...]
    for s in range(NB):
        m = b == float(s)
        v = jnp.where(m, z, NEG_INF)
        red = jnp.max(v, axis=0, keepdims=True)
        out_ref[s:s + 1, :] = jnp.maximum(out_ref[s:s + 1, :], red)


def _head1(h1, h2, h3, brow, p1a, p1b, p1c, pb1, p2, pb2, nb, *, R=512):
    N, Hh = h1.shape
    nrb = N // R
    body = functools.partial(_head1_body, R=R, Hh=Hh, NB=nb)
    return pl.pallas_call(
        body,
        grid=(nrb,),
        in_specs=[
            pl.BlockSpec((R, Hh), lambda i: (i, 0)),
            pl.BlockSpec((R, Hh), lambda i: (i, 0)),
            pl.BlockSpec((R, Hh), lambda i: (i, 0)),
            pl.BlockSpec((R, 1), lambda i: (i, 0)),
            pl.BlockSpec((Hh, Hh), lambda i: (0, 0)),
            pl.BlockSpec((Hh, Hh), lambda i: (0, 0)),
            pl.BlockSpec((Hh, Hh), lambda i: (0, 0)),
            pl.BlockSpec((1, Hh), lambda i: (0, 0)),
            pl.BlockSpec((Hh, Hh), lambda i: (0, 0)),
            pl.BlockSpec((1, Hh), lambda i: (0, 0)),
        ],
        out_specs=pl.BlockSpec((nb, Hh), lambda i: (0, 0)),
        out_shape=jax.ShapeDtypeStruct((nb, Hh), jnp.float32),
    )(h1, h2, h3, brow, p1a, p1b, p1c, pb1, p2, pb2)


def _head2_body(g_ref, m1_ref, mb1_ref, m2_ref, mb2_ref, m3_ref, mb3_ref,
                out_ref):
    g = _elu(lax.dot_general(g_ref[...], m1_ref[...], (((1,), (0,)), ((), ())),
                             preferred_element_type=jnp.float32) + mb1_ref[...])
    g = _elu(lax.dot_general(g, m2_ref[...], (((1,), (0,)), ((), ())),
                             preferred_element_type=jnp.float32) + mb2_ref[...])
    o = lax.dot_general(g, m3_ref[...], (((1,), (0,)), ((), ())),
                        preferred_element_type=jnp.float32) + mb3_ref[...]
    m = jnp.max(o, axis=1, keepdims=True)
    lse = jnp.log(jnp.sum(jnp.exp(o - m), axis=1, keepdims=True)) + m
    out_ref[...] = o - lse


def _head2(g, m1, mb1, m2, mb2, m3, mb3):
    nb, Hh = g.shape
    C = m3.shape[1]
    return pl.pallas_call(
        _head2_body,
        out_shape=jax.ShapeDtypeStruct((nb, C), jnp.float32),
    )(g, m1, mb1, m2, mb2, m3, mb3)


def kernel(x, W1a, b1a, W1b, b1b, W2a, b2a, W2b, b2b, W3a, b3a, W3b, b3b,
           P1, pb1, P2, pb2, M1, mb1, M2, mb2, M3, mb3, batch):
    N, D_in = x.shape
    Hh = W1b.shape[0]
    NB = 8
    R = 256

    bi = batch.astype(jnp.int32)
    seg_lo = jnp.searchsorted(bi, jnp.arange(NB, dtype=jnp.int32), side="left")
    seg_hi = jnp.searchsorted(bi, jnp.arange(NB, dtype=jnp.int32), side="right")
    lo = seg_lo[bi[::R]].astype(jnp.int32)
    hi = seg_hi[bi[R - 1::R]].astype(jnp.int32)

    bf = batch.astype(jnp.float32)
    brow = bf.reshape(N, 1)
    bcol = bf.reshape(1, N)

    Dp = 8
    xp = jnp.concatenate(
        [x, jnp.zeros((N, Dp - D_in), jnp.float32)], axis=1)

    def split_wa(Wa, d, dp):
        top, bot = Wa[:d], Wa[d:]
        diff = top - bot
        if dp > d:
            pad = jnp.zeros((dp - d, Wa.shape[1]), jnp.float32)
            diff = jnp.concatenate([diff, pad], axis=0)
            bot = jnp.concatenate([bot, pad], axis=0)
        return diff, bot

    h = xp
    d_cur, dp_cur = D_in, Dp
    hs = []
    for (Wa, ba, Wb, bb) in ((W1a, b1a, W1b, b1b),
                             (W2a, b2a, W2b, b2b),
                             (W3a, b3a, W3b, b3b)):
        wdiff, wbot = split_wa(Wa, d_cur, dp_cur)
        idx, a_t, g_t = _knn_and_terms(
            h, brow, bcol, lo, hi, wdiff, wbot, ba.reshape(1, Hh), R=R)
        idx_flat = idx.reshape(-1)
        xg = _sc_gather(g_t, idx_flat.reshape(-1, 128))
        xg = xg.reshape(K, N, Hh)
        h = _edge_conv(a_t, xg, Wb, bb.reshape(1, Hh), R=R)
        hs.append(h)
        d_cur = dp_cur = Hh

    p1a, p1b, p1c = P1[:Hh], P1[Hh:2 * Hh], P1[2 * Hh:]
    g = _head1(hs[0], hs[1], hs[2], brow, p1a, p1b, p1c,
               pb1.reshape(1, Hh), P2, pb2.reshape(1, Hh), NB)
    return _head2(g, M1, mb1.reshape(1, Hh), M2, mb2.reshape(1, Hh),
                  M3, mb3.reshape(1, M3.shape[1]))


# speedup vs baseline: 22.0795x; 1.1324x over previous
import functools

import jax
import jax.numpy as jnp
from jax import lax
from jax.experimental import pallas as pl
from jax.experimental.pallas import tpu as pltpu
from jax.experimental.pallas import tpu_sc as plsc

K = 16
NEG_INF = float("-inf")
FIDX_SENTINEL = 1e9


def _knn_body(lo_ref, hi_ref, hrow_ref, hcol_ref, brow_ref, bcol_ref,
              wdiff_ref, wbot_ref, ba_ref,
              idx_ref, a_ref, g_ref, bestv, besti, *, R, CB):
    r = pl.program_id(0)
    hr = hrow_ref[...]
    brt = bcol_ref[:, pl.ds(r * R, R)]

    bestv[...] = jnp.full((K, R), NEG_INF, jnp.float32)
    besti[...] = jnp.full((K, R), FIDX_SENTINEL, jnp.float32)

    lo = lo_ref[r]
    hi = hi_ref[r]
    c0 = lo // CB
    c1 = (hi + (CB - 1)) // CB

    def col_step(c, carry):
        off = c * CB
        hc = hcol_ref[pl.ds(off, CB), :]
        bc = brow_ref[pl.ds(off, CB), :]
        dot = lax.dot_general(hc, hr, (((1,), (1,)), ((), ())),
                              preferred_element_type=jnp.float32)
        sqc = jnp.sum(hc * hc, axis=1, keepdims=True)
        score = dot - 0.5 * sqc
        valid = bc == brt
        score = jnp.where(valid, score, NEG_INF)
        colidx = (jnp.float32(off)
                  + lax.broadcasted_iota(jnp.int32, (CB, R), 0).astype(jnp.float32))

        cv = jnp.concatenate([bestv[...], score], axis=0)
        ci = jnp.concatenate([besti[...], colidx], axis=0)
        for t in range(K):
            m = jnp.max(cv, axis=0, keepdims=True)
            wi = jnp.min(jnp.where(cv == m, ci, FIDX_SENTINEL), axis=0,
                         keepdims=True)
            cv = jnp.where(ci == wi, NEG_INF, cv)
            bestv[t:t + 1, :] = m
            besti[t:t + 1, :] = wi
        return carry

    lax.fori_loop(c0, c1, col_step, 0)
    idx_ref[...] = jnp.clip(besti[...], 0.0, hcol_ref.shape[0] - 1).astype(jnp.int32)

    a_ref[...] = lax.dot_general(hr, wdiff_ref[...], (((1,), (0,)), ((), ())),
                                 preferred_element_type=jnp.float32) + ba_ref[...]
    g_ref[...] = lax.dot_general(hr, wbot_ref[...], (((1,), (0,)), ((), ())),
                                 preferred_element_type=jnp.float32)


def _knn_and_terms(h, brow, bcol, lo, hi, wdiff, wbot, ba, *, R=256, CB=256):
    N, D = h.shape
    Hh = wdiff.shape[1]
    nrb = N // R
    body = functools.partial(_knn_body, R=R, CB=CB)
    return pl.pallas_call(
        body,
        grid=(nrb,),
        in_specs=[
            pl.BlockSpec(memory_space=pltpu.SMEM),
            pl.BlockSpec(memory_space=pltpu.SMEM),
            pl.BlockSpec((R, D), lambda i: (i, 0)),
            pl.BlockSpec((N, D), lambda i: (0, 0)),
            pl.BlockSpec((N, 1), lambda i: (0, 0)),
            pl.BlockSpec((1, N), lambda i: (0, 0)),
            pl.BlockSpec((D, Hh), lambda i: (0, 0)),
            pl.BlockSpec((D, Hh), lambda i: (0, 0)),
            pl.BlockSpec((1, Hh), lambda i: (0, 0)),
        ],
        out_specs=[
            pl.BlockSpec((K, R), lambda i: (0, i)),
            pl.BlockSpec((R, Hh), lambda i: (i, 0)),
            pl.BlockSpec((R, Hh), lambda i: (i, 0)),
        ],
        out_shape=[
            jax.ShapeDtypeStruct((K, N), jnp.int32),
            jax.ShapeDtypeStruct((N, Hh), jnp.float32),
            jax.ShapeDtypeStruct((N, Hh), jnp.float32),
        ],
        scratch_shapes=[
            pltpu.VMEM((K, R), jnp.float32),
            pltpu.VMEM((K, R), jnp.float32),
        ],
    )(lo, hi, h, h, brow, bcol, wdiff, wbot, ba)


def _sc_gather(table, idx2d):
    V, Hh = table.shape
    CH = idx2d.shape[1]
    E = idx2d.shape[0] * CH
    info = plsc.get_sparse_core_info()
    NW = info.num_cores * info.num_subcores
    per_w = E // NW
    nch = per_w // CH
    rows_per_w = per_w // CH

    mesh = plsc.VectorSubcoreMesh(core_axis_name="c", subcore_axis_name="s")

    @functools.partial(
        pl.kernel, mesh=mesh,
        out_type=jax.ShapeDtypeStruct((E, Hh), jnp.float32),
        scratch_types=[
            pltpu.VMEM((nch, CH), jnp.int32),
            pltpu.VMEM((CH, Hh), jnp.float32),
            pltpu.VMEM((CH, Hh), jnp.float32),
            pltpu.SemaphoreType.DMA,
            pltpu.SemaphoreType.DMA,
        ],
    )
    def gather_k(table_hbm, idx_hbm, out_hbm, idx_v, buf0, buf1, sem0, sem1):
        wid = lax.axis_index("s") * info.num_cores + lax.axis_index("c")
        base = wid * per_w
        pltpu.sync_copy(idx_hbm.at[pl.ds(wid * nch, nch)], idx_v)

        def start(j, buf, sem):
            return pltpu.async_copy(table_hbm.at[idx_v.at[j]], buf, sem)

        def wait(j, buf, sem):
            pltpu.make_async_copy(table_hbm.at[idx_v.at[j]], buf, sem).wait()

        start(0, buf0, sem0)

        def pair(p, carry):
            j0 = 2 * p
            j1 = j0 + 1
            start(j1, buf1, sem1)
            wait(j0, buf0, sem0)
            pltpu.sync_copy(buf0, out_hbm.at[pl.ds(base + j0 * CH, CH)])

            @pl.when(j1 + 1 < nch)
            def _():
                start(j1 + 1, buf0, sem0)

            wait(j1, buf1, sem1)
            pltpu.sync_copy(buf1, out_hbm.at[pl.ds(base + j1 * CH, CH)])
            return carry

        lax.fori_loop(0, nch // 2, pair, 0)

    return gather_k(table, idx2d)


def _conv_body(a_ref, xg_ref, wb_ref, bb_ref, out_ref, *, R, Hh):
    a = a_ref[...]
    wb = wb_ref[...]
    acc = jnp.full((R, Hh), NEG_INF, jnp.float32)
    for k in range(K):
        e = jnp.maximum(a + xg_ref[k], 0.0)
        mm = lax.dot_general(e, wb, (((1,), (0,)), ((), ())),
                             preferred_element_type=jnp.float32)
        acc = jnp.maximum(acc, mm)
    out_ref[...] = acc + bb_ref[...]


def _edge_conv(a, xg, wb, bb, *, R=256):
    N, Hh = a.shape
    nrb = N // R
    body = functools.partial(_conv_body, R=R, Hh=Hh)
    return pl.pallas_call(
        body,
        grid=(nrb,),
        in_specs=[
            pl.BlockSpec((R, Hh), lambda i: (i, 0)),
            pl.BlockSpec((K, R, Hh), lambda i: (0, i, 0)),
            pl.BlockSpec((Hh, Hh), lambda i: (0, 0)),
            pl.BlockSpec((1, Hh), lambda i: (0, 0)),
        ],
        out_specs=pl.BlockSpec((R, Hh), lambda i: (i, 0)),
        out_shape=jax.ShapeDtypeStruct((N, Hh), jnp.float32),
    )(a, xg, wb, bb)


def _elu(x):
    return jnp.where(x > 0, x, jnp.exp(x) - 1.0)


def _head1_body(h1_ref, h2_ref, h3_ref, brow_ref, p1a_ref, p1b_ref, p1c_ref,
                pb1_ref, p2_ref, pb2_ref, out_ref, *, R, Hh, NB):
    i = pl.program_id(0)

    @pl.when(i == 0)
    def _():
        out_ref[...] = jnp.full(out_ref.shape, NEG_INF, jnp.float32)

    z = (lax.dot_general(h1_ref[...], p1a_ref[...], (((1,), (0,)), ((), ())),
                         preferred_element_type=jnp.float32)
         + lax.dot_general(h2_ref[...], p1b_ref[...], (((1,), (0,)), ((), ())),
                           preferred_element_type=jnp.float32)
         + lax.dot_general(h3_ref[...], p1c_ref[...], (((1,), (0,)), ((), ())),
                           preferred_element_type=jnp.float32)
         + pb1_ref[...])
    z = _elu(z)
    z = _elu(lax.dot_general(z, p2_ref[...], (((1,), (0,)), ((), ())),
                             preferred_element_type=jnp.float32) + pb2_ref[...])
    b = brow_ref[...]
    for s in range(NB):
        m = b == float(s)
        v = jnp.where(m, z, NEG_INF)
        red = jnp.max(v, axis=0, keepdims=True)
        out_ref[s:s + 1, :] = jnp.maximum(out_ref[s:s + 1, :], red)


def _head1(h1, h2, h3, brow, p1a, p1b, p1c, pb1, p2, pb2, nb, *, R=512):
    N, Hh = h1.shape
    nrb = N // R
    body = functools.partial(_head1_body, R=R, Hh=Hh, NB=nb)
    return pl.pallas_call(
        body,
        grid=(nrb,),
        in_specs=[
            pl.BlockSpec((R, Hh), lambda i: (i, 0)),
            pl.BlockSpec((R, Hh), lambda i: (i, 0)),
            pl.BlockSpec((R, Hh), lambda i: (i, 0)),
            pl.BlockSpec((R, 1), lambda i: (i, 0)),
            pl.BlockSpec((Hh, Hh), lambda i: (0, 0)),
            pl.BlockSpec((Hh, Hh), lambda i: (0, 0)),
            pl.BlockSpec((Hh, Hh), lambda i: (0, 0)),
            pl.BlockSpec((1, Hh), lambda i: (0, 0)),
            pl.BlockSpec((Hh, Hh), lambda i: (0, 0)),
            pl.BlockSpec((1, Hh), lambda i: (0, 0)),
        ],
        out_specs=pl.BlockSpec((nb, Hh), lambda i: (0, 0)),
        out_shape=jax.ShapeDtypeStruct((nb, Hh), jnp.float32),
    )(h1, h2, h3, brow, p1a, p1b, p1c, pb1, p2, pb2)


def _head2_body(g_ref, m1_ref, mb1_ref, m2_ref, mb2_ref, m3_ref, mb3_ref,
                out_ref):
    g = _elu(lax.dot_general(g_ref[...], m1_ref[...], (((1,), (0,)), ((), ())),
                             preferred_element_type=jnp.float32) + mb1_ref[...])
    g = _elu(lax.dot_general(g, m2_ref[...], (((1,), (0,)), ((), ())),
                             preferred_element_type=jnp.float32) + mb2_ref[...])
    o = lax.dot_general(g, m3_ref[...], (((1,), (0,)), ((), ())),
                        preferred_element_type=jnp.float32) + mb3_ref[...]
    m = jnp.max(o, axis=1, keepdims=True)
    lse = jnp.log(jnp.sum(jnp.exp(o - m), axis=1, keepdims=True)) + m
    out_ref[...] = o - lse


def _head2(g, m1, mb1, m2, mb2, m3, mb3):
    nb, Hh = g.shape
    C = m3.shape[1]
    return pl.pallas_call(
        _head2_body,
        out_shape=jax.ShapeDtypeStruct((nb, C), jnp.float32),
    )(g, m1, mb1, m2, mb2, m3, mb3)


def kernel(x, W1a, b1a, W1b, b1b, W2a, b2a, W2b, b2b, W3a, b3a, W3b, b3b,
           P1, pb1, P2, pb2, M1, mb1, M2, mb2, M3, mb3, batch):
    N, D_in = x.shape
    Hh = W1b.shape[0]
    NB = 8
    R = 256

    bi = batch.astype(jnp.int32)
    seg_lo = jnp.searchsorted(bi, jnp.arange(NB, dtype=jnp.int32), side="left")
    seg_hi = jnp.searchsorted(bi, jnp.arange(NB, dtype=jnp.int32), side="right")
    lo = seg_lo[bi[::R]].astype(jnp.int32)
    hi = seg_hi[bi[R - 1::R]].astype(jnp.int32)

    bf = batch.astype(jnp.float32)
    brow = bf.reshape(N, 1)
    bcol = bf.reshape(1, N)

    Dp = 8
    xp = jnp.concatenate(
        [x, jnp.zeros((N, Dp - D_in), jnp.float32)], axis=1)

    def split_wa(Wa, d, dp):
        top, bot = Wa[:d], Wa[d:]
        diff = top - bot
        if dp > d:
            pad = jnp.zeros((dp - d, Wa.shape[1]), jnp.float32)
            diff = jnp.concatenate([diff, pad], axis=0)
            bot = jnp.concatenate([bot, pad], axis=0)
        return diff, bot

    h = xp
    d_cur, dp_cur = D_in, Dp
    hs = []
    for (Wa, ba, Wb, bb) in ((W1a, b1a, W1b, b1b),
                             (W2a, b2a, W2b, b2b),
                             (W3a, b3a, W3b, b3b)):
        wdiff, wbot = split_wa(Wa, d_cur, dp_cur)
        idx, a_t, g_t = _knn_and_terms(
            h, brow, bcol, lo, hi, wdiff, wbot, ba.reshape(1, Hh), R=R)
        idx_flat = idx.reshape(-1)
        xg = _sc_gather(g_t, idx_flat.reshape(-1, 128))
        xg = xg.reshape(K, N, Hh)
        h = _edge_conv(a_t, xg, Wb, bb.reshape(1, Hh), R=R)
        hs.append(h)
        d_cur = dp_cur = Hh

    p1a, p1b, p1c = P1[:Hh], P1[Hh:2 * Hh], P1[2 * Hh:]
    g = _head1(hs[0], hs[1], hs[2], brow, p1a, p1b, p1c,
               pb1.reshape(1, Hh), P2, pb2.reshape(1, Hh), NB)
    return _head2(g, M1, mb1.reshape(1, Hh), M2, mb2.reshape(1, Hh),
                  M3, mb3.reshape(1, M3.shape[1]))
